# baseline (speedup 1.0000x reference)
import functools

import jax
import jax.numpy as jnp
from jax import lax
from jax.experimental import pallas as pl
from jax.experimental.pallas import tpu as pltpu
from jax.experimental.pallas import tpu_sc as plsc

NUM_EMB = 64
EMB_DIM = 16
NC = 2
NS = 16
NW = NC * NS
LANES = 16


def _ln_table_body(t_ref, g_ref, b_ref, o_ref):
    t = t_ref[...]
    m = jnp.mean(t, axis=1, keepdims=True)
    v = jnp.mean(jnp.square(t - m), axis=1, keepdims=True)
    o_ref[...] = (t - m) / jnp.sqrt(v + 1e-5) * g_ref[...] + b_ref[...]


def _ln_table(table, gamma, beta):
    return pl.pallas_call(
        _ln_table_body,
        out_shape=jax.ShapeDtypeStruct(table.shape, table.dtype),
    )(table, gamma, beta)


def _make_expand(B):
    assert B % (8 * NW) == 0
    bpw = B // NW
    C = 800
    NB = 2
    assert bpw % (C * NB) == 0
    npairs = bpw // (C * NB)
    mesh = plsc.VectorSubcoreMesh(core_axis_name="c", subcore_axis_name="s")

    @functools.partial(
        pl.kernel,
        out_type=jax.ShapeDtypeStruct((B * EMB_DIM,), jnp.float32),
        mesh=mesh,
        scratch_types=[
            pltpu.VMEM((NUM_EMB * EMB_DIM,), jnp.float32),
            pltpu.VMEM((bpw,), jnp.int32),
            pltpu.VMEM((NB, C * EMB_DIM), jnp.float32),
            pltpu.SemaphoreType.DMA,
            pltpu.SemaphoreType.DMA,
        ],
        compiler_params=pltpu.CompilerParams(
            use_tc_tiling_on_sc=False, needs_layout_passes=False),
    )
    def expand(tab_hbm, idx_hbm, out_hbm, tab_v, idx_v, rows_v, s0, s1):
        ssems = (s0, s1)
        wid = lax.axis_index("s") * NC + lax.axis_index("c")
        base = wid * bpw
        pltpu.sync_copy(tab_hbm, tab_v)
        pltpu.sync_copy(idx_hbm.at[pl.ds(base, bpw)], idx_v)

        iota = lax.iota(jnp.int32, LANES)
        _dnums = lax.GatherDimensionNumbers(
            offset_dims=(), collapsed_slice_dims=(0,), start_index_map=(0,))

        def lane_bcast(v, j):
            ids = jnp.full((LANES, 1), j, jnp.int32)
            return lax.gather(v, ids, _dnums, (1,),
                              mode=lax.GatherScatterMode.PROMISE_IN_BOUNDS)

        def s_copy(i, b):
            return pltpu.make_async_copy(
                rows_v.at[b],
                out_hbm.at[pl.ds((base + i * C) * EMB_DIM, C * EMB_DIM)],
                ssems[b])

        def compute(i, b):
            def group(g, carry):
                v = idx_v[pl.ds(i * C + g * LANES, LANES)]
                v16 = v * EMB_DIM
                for j in range(LANES):
                    bj = lane_bcast(v16, j)
                    row = plsc.load_gather(tab_v, [bj + iota])
                    rows_v[b, pl.ds((g * LANES + j) * EMB_DIM, EMB_DIM)] = row
                return carry

            lax.fori_loop(0, C // LANES, group, 0)

        def body(jp, carry):
            for b in range(NB):
                i = jp * NB + b

                @pl.when(jp >= 1)
                def _():
                    s_copy(i - NB, b).wait()

                compute(i, b)
                s_copy(i, b).start()
            return carry

        lax.fori_loop(0, npairs, body, 0)
        for b in range(NB):
            s_copy((npairs - 1) * NB + b, b).wait()

    return expand


def kernel(x, table, gamma, beta):
    Bx, L = x.shape
    normed = _ln_table(table, gamma.reshape(1, EMB_DIM), beta.reshape(1, EMB_DIM))
    flat = x.reshape(-1)
    out = _make_expand(flat.shape[0])(normed.reshape(-1), flat)
    return out.reshape(Bx, L, EMB_DIM)
